# Initial kernel scaffold; baseline (speedup 1.0000x reference)
import jax
import jax.numpy as jnp
from jax.experimental import pallas as pl


def kernel(indices, embeddings):
    raise NotImplementedError("write your pallas kernel here")


# speedup vs baseline: 1.6839x; 1.6839x over previous
import functools

import jax
import jax.numpy as jnp
from jax import lax
from jax.experimental import pallas as pl
from jax.experimental.pallas import tpu as pltpu
from jax.experimental.pallas import tpu_sc as plsc

_NC = 2
_NS = 16
_NW = _NC * _NS
_G = 128


def _gather_call(num_rows, hidden, n_gathers):
    mesh = plsc.VectorSubcoreMesh(core_axis_name="c", subcore_axis_name="s")
    per_w = n_gathers * _G

    @functools.partial(
        pl.kernel,
        mesh=mesh,
        compiler_params=pltpu.CompilerParams(use_tc_tiling_on_sc=False),
        out_type=jax.ShapeDtypeStruct((_NW * per_w, hidden), jnp.float32),
        scratch_types=[
            pltpu.VMEM((n_gathers, _G), jnp.int32),
            pltpu.VMEM((_G, hidden), jnp.float32),
            pltpu.SemaphoreType.DMA,
        ],
    )
    def run(idx_hbm, tab_hbm, out_hbm, idx_v, rows_v, gsem):
        wid = lax.axis_index("s") * _NC + lax.axis_index("c")
        pltpu.sync_copy(idx_hbm.at[wid], idx_v)
        base = wid * per_w

        def step(j, carry):
            pltpu.async_copy(tab_hbm.at[idx_v.at[j]], rows_v, gsem).wait()
            pltpu.sync_copy(rows_v, out_hbm.at[pl.ds(base + j * _G, _G)])
            return carry

        lax.fori_loop(0, n_gathers, step, 0)

    return run


def kernel(indices, embeddings):
    batch, hist = indices.shape
    num_rows, hidden = embeddings.shape
    n = batch * hist
    per_w = n // _NW
    n_gathers = per_w // _G
    idx_rs = indices.astype(jnp.int32).reshape(_NW, n_gathers, _G)
    out = _gather_call(num_rows, hidden, n_gathers)(idx_rs, embeddings)
    return out.reshape(batch, hist, hidden)


# speedup vs baseline: 1.8691x; 1.1100x over previous
import functools

import jax
import jax.numpy as jnp
from jax import lax
from jax.experimental import pallas as pl
from jax.experimental.pallas import tpu as pltpu
from jax.experimental.pallas import tpu_sc as plsc

_NC = 2
_NS = 16
_NW = _NC * _NS
_G = 128
_K = 4
_NB = 3


def _gather_call(num_rows, hidden, n_groups):
    mesh = plsc.VectorSubcoreMesh(core_axis_name="c", subcore_axis_name="s")
    per_w = n_groups * _K * _G

    @functools.partial(
        pl.kernel,
        mesh=mesh,
        compiler_params=pltpu.CompilerParams(use_tc_tiling_on_sc=False),
        out_type=jax.ShapeDtypeStruct((_NW * per_w, hidden), jnp.float32),
        scratch_types=[
            pltpu.VMEM((n_groups * _K, _G), jnp.int32),
            pltpu.VMEM((_NB, _K, _G, hidden), jnp.float32),
            pltpu.SemaphoreType.DMA,
            pltpu.SemaphoreType.DMA,
            pltpu.SemaphoreType.DMA,
            pltpu.SemaphoreType.DMA,
            pltpu.SemaphoreType.DMA,
            pltpu.SemaphoreType.DMA,
        ],
    )
    def run(idx_hbm, tab_hbm, out_hbm, idx_v, banks, g0, g1, g2, s0, s1, s2):
        gsem = (g0, g1, g2)
        ssem = (s0, s1, s2)
        wid = lax.axis_index("s") * _NC + lax.axis_index("c")
        pltpu.sync_copy(idx_hbm.at[wid], idx_v)
        base = wid * per_w

        def fire_gathers(g, p):
            for k in range(_K):
                j = g * _K + k
                pltpu.async_copy(tab_hbm.at[idx_v.at[j]], banks.at[p, k], gsem[p])

        def drain_gathers(p):
            for k in range(_K):
                pltpu.make_async_copy(
                    tab_hbm.at[idx_v.at[0]], banks.at[p, k], gsem[p]
                ).wait()

        def fire_stores(g, p):
            for k in range(_K):
                j = g * _K + k
                pltpu.async_copy(
                    banks.at[p, k], out_hbm.at[pl.ds(base + j * _G, _G)], ssem[p]
                )

        def drain_stores(p):
            for k in range(_K):
                pltpu.make_async_copy(
                    banks.at[p, k], out_hbm.at[pl.ds(base, _G)], ssem[p]
                ).wait()

        def step(g, p, fire_next, drain_prev):
            if drain_prev:
                drain_stores((p + 1) % _NB)
            if fire_next:
                fire_gathers(g + 1, (p + 1) % _NB)
            drain_gathers(p)
            fire_stores(g, p)

        fire_gathers(0, 0)
        step(0, 0, True, False)
        step(1, 1, True, False)
        step(2, 2, True, True)

        def body(i, carry):
            g = i * _NB
            step(g, 0, True, True)
            step(g + 1, 1, True, True)
            step(g + 2, 2, True, True)
            return carry

        lax.fori_loop(1, (n_groups - 2) // _NB, body, 0)
        step(n_groups - 2, (n_groups - 2) % _NB, True, True)
        step(n_groups - 1, (n_groups - 1) % _NB, False, True)
        drain_stores((n_groups - 2) % _NB)
        drain_stores((n_groups - 1) % _NB)

    return run


def kernel(indices, embeddings):
    batch, hist = indices.shape
    num_rows, hidden = embeddings.shape
    n = batch * hist
    per_w = n // _NW
    n_groups = per_w // (_K * _G)
    idx_rs = indices.astype(jnp.int32).reshape(_NW, n_groups * _K, _G)
    out = _gather_call(num_rows, hidden, n_groups)(idx_rs, embeddings)
    return out.reshape(batch, hist, hidden)
